# baseline (speedup 1.0000x reference)
import functools
from typing import Any

import jax
import jax.numpy as jnp
from jax import lax
from jax.experimental import pallas as pl
from jax.experimental.pallas import tpu as pltpu
from jax.experimental.pallas import tpu_sc as plsc

D_MODEL = 256
N_HEADS = 8
N_LEVELS = 4
N_POINTS = 4
SPATIAL = [(64, 64), (32, 32), (16, 16), (8, 8)]
SEQ = sum(h * w for h, w in SPATIAL)
DH = D_MODEL // N_HEADS
HK = N_HEADS * 2

QPAD = 5504
QBLK_B = QPAD
SBLK_A = 544
QBLK_D = QPAD
QC = 128
NSUB = QC // 16


def _value_proj_body(enc_ref, wv_ref, bv_ref, out_ref):
    x = enc_ref[0]
    out_ref[0] = jnp.dot(x, wv_ref[...], preferred_element_type=jnp.float32,
                         precision=jax.lax.Precision.HIGHEST) + bv_ref[...]


def _value_proj(enc, W_value, b_value):
    B, S, d = enc.shape
    grid = (B, S // SBLK_A)
    return pl.pallas_call(
        _value_proj_body,
        grid=grid,
        in_specs=[
            pl.BlockSpec((1, SBLK_A, d), lambda b, s: (b, s, 0)),
            pl.BlockSpec((d, d), lambda b, s: (0, 0)),
            pl.BlockSpec((d,), lambda b, s: (0,)),
        ],
        out_specs=pl.BlockSpec((1, SBLK_A, d), lambda b, s: (b, s, 0)),
        out_shape=jax.ShapeDtypeStruct((B, S, d), jnp.float32),
    )(enc, W_value, b_value)


def _sample_body(hsT_ref, refT_ref, woff_ref, boff_ref, wattn_ref, battn_ref,
                 wl_ref, hl_ref, ls_ref,
                 attn_ref, idx_ref, w_ref):
    hsT = hsT_ref[0]
    wo = woff_ref[0]
    bo = boff_ref[0]
    hp = jax.lax.Precision.HIGHEST
    offx = jnp.dot(wo[0], hsT, preferred_element_type=jnp.float32, precision=hp) + bo[0]
    offy = jnp.dot(wo[1], hsT, preferred_element_type=jnp.float32, precision=hp) + bo[1]
    z = jnp.dot(wattn_ref[0], hsT, preferred_element_type=jnp.float32, precision=hp) + battn_ref[0]
    z = z - jnp.max(z, axis=0, keepdims=True)
    e = jnp.exp(z)
    a = e / jnp.sum(e, axis=0, keepdims=True)
    attn_ref[0, 0] = a
    rx = refT_ref[0, 0]
    ry = refT_ref[0, 1]
    wl = wl_ref[...]
    hl = hl_ref[...]
    ls = ls_ref[...]
    x = rx * wl + offx - 0.5
    y = ry * hl + offy - 0.5
    x0 = jnp.floor(x)
    y0 = jnp.floor(y)
    idxs = []
    wgts = []
    for dy in (0, 1):
        for dx in (0, 1):
            xi = x0 + dx
            yi = y0 + dy
            wxy = (1.0 - jnp.abs(x - xi)) * (1.0 - jnp.abs(y - yi))
            valid = ((xi >= 0) & (xi <= wl - 1) & (yi >= 0) & (yi <= hl - 1))
            xc = jnp.clip(xi, 0, wl - 1)
            yc = jnp.clip(yi, 0, hl - 1)
            row = (ls + yc * wl + xc) * 16.0
            idxs.append(row.astype(jnp.int32))
            wgts.append(jnp.where(valid, wxy, 0.0) * a)
    idx_ref[0, 0] = jnp.stack(idxs)
    w_ref[0, 0] = jnp.stack(wgts)


def _build_samples(hsT, refT, W_offT, b_offT, W_attnT, b_attnT,
                   wl, hl, ls):
    B = hsT.shape[0]
    Q = hsT.shape[2]
    nqb = Q // QBLK_B
    grid = (B, nqb, N_HEADS)
    return pl.pallas_call(
        _sample_body,
        grid=grid,
        in_specs=[
            pl.BlockSpec((1, D_MODEL, QBLK_B), lambda b, q, h: (b, 0, q)),
            pl.BlockSpec((1, 2, 16, QBLK_B), lambda b, q, h: (b, 0, 0, q)),
            pl.BlockSpec((1, 2, 16, D_MODEL), lambda b, q, h: (h, 0, 0, 0)),
            pl.BlockSpec((1, 2, 16, 1), lambda b, q, h: (h, 0, 0, 0)),
            pl.BlockSpec((1, 16, D_MODEL), lambda b, q, h: (h, 0, 0)),
            pl.BlockSpec((1, 16, 1), lambda b, q, h: (h, 0, 0)),
            pl.BlockSpec((16, 1), lambda b, q, h: (0, 0)),
            pl.BlockSpec((16, 1), lambda b, q, h: (0, 0)),
            pl.BlockSpec((16, 1), lambda b, q, h: (0, 0)),
        ],
        out_specs=[
            pl.BlockSpec((1, 1, 16, QBLK_B), lambda b, q, h: (b, h, 0, q)),
            pl.BlockSpec((1, 1, 4, 16, QBLK_B), lambda b, q, h: (b, h, 0, 0, q)),
            pl.BlockSpec((1, 1, 4, 16, QBLK_B), lambda b, q, h: (b, h, 0, 0, q)),
        ],
        out_shape=[
            jax.ShapeDtypeStruct((B, N_HEADS, 16, Q), jnp.float32),
            jax.ShapeDtypeStruct((B, N_HEADS, 4, 16, Q), jnp.int32),
            jax.ShapeDtypeStruct((B, N_HEADS, 4, 16, Q), jnp.float32),
        ],
    )(hsT, refT, W_offT, b_offT, W_attnT, b_attnT, wl, hl, ls)


def _sc_gather(value4, idx4, w4):
    B = value4.shape[0]
    S = value4.shape[2] // 16
    Q = idx4.shape[-1]
    assert Q % QC == 0
    nfull = Q // QC
    mesh = plsc.VectorSubcoreMesh(core_axis_name="c", subcore_axis_name="s")

    @functools.partial(
        pl.kernel,
        out_type=jax.ShapeDtypeStruct((B, HK, 16, Q), jnp.float32),
        mesh=mesh,
        compiler_params=pltpu.CompilerParams(needs_layout_passes=False),
        scratch_types=[
            pltpu.VMEM((S * 16,), jnp.float32),
            pltpu.VMEM((4, 16, QC), jnp.int32),
            pltpu.VMEM((4, 16, QC), jnp.float32),
            pltpu.VMEM((16, QC), jnp.float32),
            pltpu.SemaphoreType.DMA,
        ],
    )
    def body(value_hbm, idx_hbm, w_hbm, out_hbm, val_v, idx_v, w_v, out_v, sem):
        b = lax.axis_index("c")
        s = lax.axis_index("s")
        h = s // 2
        pltpu.sync_copy(value_hbm.at[b, s], val_v)

        def subblock(j):
            js = j * 16
            for half in range(2):
                acc = [jnp.zeros((16,), jnp.float32) for _ in range(8)]
                for c in range(4):
                    for t in range(16):
                        rows = idx_v[c, t, pl.ds(js, 16)]
                        wv = w_v[c, t, pl.ds(js, 16)]
                        for e in range(8):
                            d = half * 8 + e
                            g = plsc.load_gather(val_v, [rows + jnp.int32(d)])
                            acc[e] = acc[e] + wv * g
                for e in range(8):
                    out_v[half * 8 + e, pl.ds(js, 16)] = acc[e]

        def chunk_body(k):
            q0 = k * QC
            pltpu.sync_copy(idx_hbm.at[b, h, :, :, pl.ds(q0, QC)], idx_v)
            pltpu.sync_copy(w_hbm.at[b, h, :, :, pl.ds(q0, QC)], w_v)
            pl.loop(0, NSUB)(subblock)
            pltpu.sync_copy(out_v, out_hbm.at[b, s, :, pl.ds(q0, QC)])

        pl.loop(0, nfull)(chunk_body)

    return body(value4, idx4, w4)


def _out_proj_body(sc_ref, wo_ref, bo_ref, out_ref):
    x = sc_ref[0].reshape(D_MODEL, QBLK_D)
    y = lax.dot_general(x, wo_ref[...], (((0,), (0,)), ((), ())),
                        preferred_element_type=jnp.float32,
                        precision=jax.lax.Precision.HIGHEST)
    out_ref[0] = y + bo_ref[...]


def _out_proj(sc_out, W_out, b_out):
    B = sc_out.shape[0]
    Q = sc_out.shape[3]
    assert Q == QBLK_D
    grid = (B, Q // QBLK_D)
    return pl.pallas_call(
        _out_proj_body,
        grid=grid,
        in_specs=[
            pl.BlockSpec((1, HK, 16, QBLK_D), lambda b, q: (b, 0, 0, q)),
            pl.BlockSpec((D_MODEL, D_MODEL), lambda b, q: (0, 0)),
            pl.BlockSpec((D_MODEL,), lambda b, q: (0,)),
        ],
        out_specs=pl.BlockSpec((1, QBLK_D, D_MODEL), lambda b, q: (b, q, 0)),
        out_shape=jax.ShapeDtypeStruct((B, Q, D_MODEL), jnp.float32),
    )(sc_out, W_out, b_out)


def kernel(hidden_states, encoder_hidden_states, reference_points,
           spatial_shapes, level_start_index, W_value, b_value, W_off, b_off,
           W_attn, b_attn, W_out, b_out):
    B, Q, d = hidden_states.shape
    S = encoder_hidden_states.shape[1]
    H, L, P = N_HEADS, N_LEVELS, N_POINTS

    pad = QPAD - Q
    hsT = jnp.swapaxes(hidden_states, 1, 2)
    hsT = jnp.pad(hsT, ((0, 0), (0, 0), (0, pad)))
    refT = jnp.transpose(reference_points, (0, 3, 2, 1))
    refT = jnp.repeat(refT, N_POINTS, axis=2)
    refT = jnp.pad(refT, ((0, 0), (0, 0), (0, 0), (0, pad)))
    W_offT = jnp.transpose(W_off.reshape(d, H, 16, 2), (1, 3, 2, 0))
    b_offT = jnp.transpose(b_off.reshape(H, 16, 2), (0, 2, 1))[..., None]
    W_attnT = jnp.transpose(W_attn.reshape(d, H, 16), (1, 2, 0))
    b_attnT = b_attn.reshape(H, 16)[..., None]

    lp = jnp.arange(16)
    lvl = lp // N_POINTS
    sh = jnp.array(SPATIAL, jnp.float32)
    wl = sh[:, 1][lvl][:, None]
    hl = sh[:, 0][lvl][:, None]
    sizes = jnp.array([hh * ww for hh, ww in SPATIAL], jnp.float32)
    starts = jnp.concatenate([jnp.zeros((1,), jnp.float32), jnp.cumsum(sizes)[:-1]])
    ls = starts[lvl][:, None]

    value = _value_proj(encoder_hidden_states, W_value, b_value)
    value4 = jnp.transpose(value.reshape(B, S, HK, 16), (0, 2, 1, 3)).reshape(B, HK, S * 16)

    attnT, idx4, w4 = _build_samples(hsT, refT, W_offT, b_offT,
                                     W_attnT, b_attnT, wl, hl, ls)

    sc_out = _sc_gather(value4, idx4, w4)

    out = _out_proj(sc_out, W_out, b_out)[:, :Q]

    attn = jnp.transpose(attnT[..., :Q], (0, 3, 1, 2)).reshape(B, Q, H, L, P)
    return out, attn


# speedup vs baseline: 1.2705x; 1.2705x over previous
import functools
from typing import Any

import jax
import jax.numpy as jnp
from jax import lax
from jax.experimental import pallas as pl
from jax.experimental.pallas import tpu as pltpu
from jax.experimental.pallas import tpu_sc as plsc

D_MODEL = 256
N_HEADS = 8
N_LEVELS = 4
N_POINTS = 4
SPATIAL = [(64, 64), (32, 32), (16, 16), (8, 8)]
SEQ = sum(h * w for h, w in SPATIAL)
DH = D_MODEL // N_HEADS
HK = N_HEADS * 2

QPAD = 5504
QBLK_B = QPAD
SBLK_A = 544
QBLK_D = QPAD
QC = 128
NSUB = QC // 16


def _value_proj_body(enc_ref, wv_ref, bv_ref, out_ref):
    x = enc_ref[0]
    out_ref[0] = jnp.dot(x, wv_ref[...], preferred_element_type=jnp.float32,
                         precision=jax.lax.Precision.HIGHEST) + bv_ref[...]


def _value_proj(enc, W_value, b_value):
    B, S, d = enc.shape
    grid = (B, S // SBLK_A)
    return pl.pallas_call(
        _value_proj_body,
        grid=grid,
        in_specs=[
            pl.BlockSpec((1, SBLK_A, d), lambda b, s: (b, s, 0)),
            pl.BlockSpec((d, d), lambda b, s: (0, 0)),
            pl.BlockSpec((d,), lambda b, s: (0,)),
        ],
        out_specs=pl.BlockSpec((1, SBLK_A, d), lambda b, s: (b, s, 0)),
        out_shape=jax.ShapeDtypeStruct((B, S, d), jnp.float32),
    )(enc, W_value, b_value)


def _sample_body(hsT_ref, refT_ref, woff_ref, boff_ref, wattn_ref, battn_ref,
                 wl_ref, hl_ref, ls_ref,
                 attn_ref, idx_ref, w_ref):
    hsT = hsT_ref[0]
    wo = woff_ref[0]
    bo = boff_ref[0]
    hp = jax.lax.Precision.HIGHEST
    offx = jnp.dot(wo[0], hsT, preferred_element_type=jnp.float32, precision=hp) + bo[0]
    offy = jnp.dot(wo[1], hsT, preferred_element_type=jnp.float32, precision=hp) + bo[1]
    z = jnp.dot(wattn_ref[0], hsT, preferred_element_type=jnp.float32, precision=hp) + battn_ref[0]
    z = z - jnp.max(z, axis=0, keepdims=True)
    e = jnp.exp(z)
    a = e / jnp.sum(e, axis=0, keepdims=True)
    attn_ref[0, 0] = a
    rx = refT_ref[0, 0]
    ry = refT_ref[0, 1]
    wl = wl_ref[...]
    hl = hl_ref[...]
    ls = ls_ref[...]
    x = rx * wl + offx - 0.5
    y = ry * hl + offy - 0.5
    x0 = jnp.floor(x)
    y0 = jnp.floor(y)
    idxs = []
    wgts = []
    for dy in (0, 1):
        for dx in (0, 1):
            xi = x0 + dx
            yi = y0 + dy
            wxy = (1.0 - jnp.abs(x - xi)) * (1.0 - jnp.abs(y - yi))
            valid = ((xi >= 0) & (xi <= wl - 1) & (yi >= 0) & (yi <= hl - 1))
            xc = jnp.clip(xi, 0, wl - 1)
            yc = jnp.clip(yi, 0, hl - 1)
            row = ls + yc * wl + xc
            idxs.append(row.astype(jnp.int32))
            wgts.append(jnp.where(valid, wxy, 0.0) * a)
    idx_ref[0, 0] = jnp.stack(idxs)
    w_ref[0, 0] = jnp.stack(wgts)


def _build_samples(hsT, refT, W_offT, b_offT, W_attnT, b_attnT,
                   wl, hl, ls):
    B = hsT.shape[0]
    Q = hsT.shape[2]
    nqb = Q // QBLK_B
    grid = (B, nqb, N_HEADS)
    return pl.pallas_call(
        _sample_body,
        grid=grid,
        in_specs=[
            pl.BlockSpec((1, D_MODEL, QBLK_B), lambda b, q, h: (b, 0, q)),
            pl.BlockSpec((1, 2, 16, QBLK_B), lambda b, q, h: (b, 0, 0, q)),
            pl.BlockSpec((1, 2, 16, D_MODEL), lambda b, q, h: (h, 0, 0, 0)),
            pl.BlockSpec((1, 2, 16, 1), lambda b, q, h: (h, 0, 0, 0)),
            pl.BlockSpec((1, 16, D_MODEL), lambda b, q, h: (h, 0, 0)),
            pl.BlockSpec((1, 16, 1), lambda b, q, h: (h, 0, 0)),
            pl.BlockSpec((16, 1), lambda b, q, h: (0, 0)),
            pl.BlockSpec((16, 1), lambda b, q, h: (0, 0)),
            pl.BlockSpec((16, 1), lambda b, q, h: (0, 0)),
        ],
        out_specs=[
            pl.BlockSpec((1, 1, 16, QBLK_B), lambda b, q, h: (b, h, 0, q)),
            pl.BlockSpec((1, 1, 4, 16, QBLK_B), lambda b, q, h: (b, h, 0, 0, q)),
            pl.BlockSpec((1, 1, 4, 16, QBLK_B), lambda b, q, h: (b, h, 0, 0, q)),
        ],
        out_shape=[
            jax.ShapeDtypeStruct((B, N_HEADS, 16, Q), jnp.float32),
            jax.ShapeDtypeStruct((B, N_HEADS, 4, 16, Q), jnp.int32),
            jax.ShapeDtypeStruct((B, N_HEADS, 4, 16, Q), jnp.float32),
        ],
    )(hsT, refT, W_offT, b_offT, W_attnT, b_attnT, wl, hl, ls)


def _sc_gather(value4, idx4, w4):
    B = value4.shape[0]
    S = value4.shape[2] // 16
    Q = idx4.shape[-1]
    assert Q % QC == 0
    nfull = Q // QC
    mesh = plsc.VectorSubcoreMesh(core_axis_name="c", subcore_axis_name="s")

    @functools.partial(
        pl.kernel,
        out_type=jax.ShapeDtypeStruct((B, HK, 16, Q), jnp.float32),
        mesh=mesh,
        compiler_params=pltpu.CompilerParams(needs_layout_passes=False),
        scratch_types=[
            pltpu.VMEM((16 * S,), jnp.float32),
            pltpu.VMEM((4, 16, QC), jnp.int32),
            pltpu.VMEM((4, 16, QC), jnp.float32),
            pltpu.VMEM((16, QC), jnp.float32),
            pltpu.SemaphoreType.DMA,
        ],
    )
    def body(value_hbm, idx_hbm, w_hbm, out_hbm, val_v, idx_v, w_v, out_v, sem):
        b = lax.axis_index("c")
        s = lax.axis_index("s")
        h = s // 2
        pltpu.sync_copy(value_hbm.at[b, s], val_v)
        views = [val_v.at[pl.ds(d * S, S)] for d in range(16)]

        def subblock(j):
            js = j * 16
            acc = [jnp.zeros((16,), jnp.float32) for _ in range(16)]
            for c in range(4):
                for t in range(16):
                    rows = idx_v[c, t, pl.ds(js, 16)]
                    wv = w_v[c, t, pl.ds(js, 16)]
                    for d in range(16):
                        g = plsc.load_gather(views[d], [rows])
                        acc[d] = acc[d] + wv * g
            for d in range(16):
                out_v[d, pl.ds(js, 16)] = acc[d]

        def chunk_body(k):
            q0 = k * QC
            pltpu.sync_copy(idx_hbm.at[b, h, :, :, pl.ds(q0, QC)], idx_v)
            pltpu.sync_copy(w_hbm.at[b, h, :, :, pl.ds(q0, QC)], w_v)
            pl.loop(0, NSUB)(subblock)
            pltpu.sync_copy(out_v, out_hbm.at[b, s, :, pl.ds(q0, QC)])

        pl.loop(0, nfull)(chunk_body)

    return body(value4, idx4, w4)


def _out_proj_body(sc_ref, wo_ref, bo_ref, out_ref):
    x = sc_ref[0].reshape(D_MODEL, QBLK_D)
    y = lax.dot_general(x, wo_ref[...], (((0,), (0,)), ((), ())),
                        preferred_element_type=jnp.float32,
                        precision=jax.lax.Precision.HIGHEST)
    out_ref[0] = y + bo_ref[...]


def _out_proj(sc_out, W_out, b_out):
    B = sc_out.shape[0]
    Q = sc_out.shape[3]
    assert Q == QBLK_D
    grid = (B, Q // QBLK_D)
    return pl.pallas_call(
        _out_proj_body,
        grid=grid,
        in_specs=[
            pl.BlockSpec((1, HK, 16, QBLK_D), lambda b, q: (b, 0, 0, q)),
            pl.BlockSpec((D_MODEL, D_MODEL), lambda b, q: (0, 0)),
            pl.BlockSpec((D_MODEL,), lambda b, q: (0,)),
        ],
        out_specs=pl.BlockSpec((1, QBLK_D, D_MODEL), lambda b, q: (b, q, 0)),
        out_shape=jax.ShapeDtypeStruct((B, Q, D_MODEL), jnp.float32),
    )(sc_out, W_out, b_out)


def kernel(hidden_states, encoder_hidden_states, reference_points,
           spatial_shapes, level_start_index, W_value, b_value, W_off, b_off,
           W_attn, b_attn, W_out, b_out):
    B, Q, d = hidden_states.shape
    S = encoder_hidden_states.shape[1]
    H, L, P = N_HEADS, N_LEVELS, N_POINTS

    pad = QPAD - Q
    hsT = jnp.swapaxes(hidden_states, 1, 2)
    hsT = jnp.pad(hsT, ((0, 0), (0, 0), (0, pad)))
    refT = jnp.transpose(reference_points, (0, 3, 2, 1))
    refT = jnp.repeat(refT, N_POINTS, axis=2)
    refT = jnp.pad(refT, ((0, 0), (0, 0), (0, 0), (0, pad)))
    W_offT = jnp.transpose(W_off.reshape(d, H, 16, 2), (1, 3, 2, 0))
    b_offT = jnp.transpose(b_off.reshape(H, 16, 2), (0, 2, 1))[..., None]
    W_attnT = jnp.transpose(W_attn.reshape(d, H, 16), (1, 2, 0))
    b_attnT = b_attn.reshape(H, 16)[..., None]

    lp = jnp.arange(16)
    lvl = lp // N_POINTS
    sh = jnp.array(SPATIAL, jnp.float32)
    wl = sh[:, 1][lvl][:, None]
    hl = sh[:, 0][lvl][:, None]
    sizes = jnp.array([hh * ww for hh, ww in SPATIAL], jnp.float32)
    starts = jnp.concatenate([jnp.zeros((1,), jnp.float32), jnp.cumsum(sizes)[:-1]])
    ls = starts[lvl][:, None]

    value = _value_proj(encoder_hidden_states, W_value, b_value)
    value4 = jnp.transpose(value.reshape(B, S, HK, 16), (0, 2, 3, 1)).reshape(B, HK, 16 * S)

    attnT, idx4, w4 = _build_samples(hsT, refT, W_offT, b_offT,
                                     W_attnT, b_attnT, wl, hl, ls)

    sc_out = _sc_gather(value4, idx4, w4)

    out = _out_proj(sc_out, W_out, b_out)[:, :Q]

    attn = jnp.transpose(attnT[..., :Q], (0, 3, 1, 2)).reshape(B, Q, H, L, P)
    return out, attn


# speedup vs baseline: 1.2913x; 1.0164x over previous
import functools
from typing import Any

import jax
import jax.numpy as jnp
from jax import lax
from jax.experimental import pallas as pl
from jax.experimental.pallas import tpu as pltpu
from jax.experimental.pallas import tpu_sc as plsc

D_MODEL = 256
N_HEADS = 8
N_LEVELS = 4
N_POINTS = 4
SPATIAL = [(64, 64), (32, 32), (16, 16), (8, 8)]
SEQ = sum(h * w for h, w in SPATIAL)
DH = D_MODEL // N_HEADS
HK = N_HEADS * 2

QPAD = 5504
QBLK_B = QPAD
SBLK_A = 544
QBLK_D = QPAD
QC = 128
NSUB = QC // 16


def _value_proj_body(enc_ref, wv_ref, bv_ref, out_ref):
    x = enc_ref[0]
    out_ref[0] = jnp.dot(x, wv_ref[...], preferred_element_type=jnp.float32,
                         precision=jax.lax.Precision.HIGHEST) + bv_ref[...]


def _value_proj(enc, W_value, b_value):
    B, S, d = enc.shape
    grid = (B, S // SBLK_A)
    return pl.pallas_call(
        _value_proj_body,
        grid=grid,
        in_specs=[
            pl.BlockSpec((1, SBLK_A, d), lambda b, s: (b, s, 0)),
            pl.BlockSpec((d, d), lambda b, s: (0, 0)),
            pl.BlockSpec((d,), lambda b, s: (0,)),
        ],
        out_specs=pl.BlockSpec((1, SBLK_A, d), lambda b, s: (b, s, 0)),
        out_shape=jax.ShapeDtypeStruct((B, S, d), jnp.float32),
    )(enc, W_value, b_value)


def _sample_body(hsT_ref, refT_ref, woff_ref, boff_ref, wattn_ref, battn_ref,
                 wl_ref, hl_ref, ls_ref,
                 attn_ref, idx_ref, w_ref):
    hsT = hsT_ref[0]
    wo = woff_ref[0]
    bo = boff_ref[0]
    hp = jax.lax.Precision.HIGHEST
    offx = jnp.dot(wo[0], hsT, preferred_element_type=jnp.float32, precision=hp) + bo[0]
    offy = jnp.dot(wo[1], hsT, preferred_element_type=jnp.float32, precision=hp) + bo[1]
    z = jnp.dot(wattn_ref[0], hsT, preferred_element_type=jnp.float32, precision=hp) + battn_ref[0]
    z = z - jnp.max(z, axis=0, keepdims=True)
    e = jnp.exp(z)
    a = e / jnp.sum(e, axis=0, keepdims=True)
    attn_ref[0, 0] = a
    rx = refT_ref[0, 0]
    ry = refT_ref[0, 1]
    wl = wl_ref[...]
    hl = hl_ref[...]
    ls = ls_ref[...]
    x = rx * wl + offx - 0.5
    y = ry * hl + offy - 0.5
    x0 = jnp.floor(x)
    y0 = jnp.floor(y)
    idxs = []
    wgts = []
    for dy in (0, 1):
        for dx in (0, 1):
            xi = x0 + dx
            yi = y0 + dy
            wxy = (1.0 - jnp.abs(x - xi)) * (1.0 - jnp.abs(y - yi))
            valid = ((xi >= 0) & (xi <= wl - 1) & (yi >= 0) & (yi <= hl - 1))
            xc = jnp.clip(xi, 0, wl - 1)
            yc = jnp.clip(yi, 0, hl - 1)
            row = ls + yc * wl + xc
            idxs.append(row.astype(jnp.int32))
            wgts.append(jnp.where(valid, wxy, 0.0) * a)
    idx_ref[0, 0] = jnp.stack(idxs)
    w_ref[0, 0] = jnp.stack(wgts)


def _build_samples(hsT, refT, W_offT, b_offT, W_attnT, b_attnT,
                   wl, hl, ls):
    B = hsT.shape[0]
    Q = hsT.shape[2]
    nqb = Q // QBLK_B
    grid = (B, nqb, N_HEADS)
    return pl.pallas_call(
        _sample_body,
        grid=grid,
        in_specs=[
            pl.BlockSpec((1, D_MODEL, QBLK_B), lambda b, q, h: (b, 0, q)),
            pl.BlockSpec((1, 2, 16, QBLK_B), lambda b, q, h: (b, 0, 0, q)),
            pl.BlockSpec((1, 2, 16, D_MODEL), lambda b, q, h: (h, 0, 0, 0)),
            pl.BlockSpec((1, 2, 16, 1), lambda b, q, h: (h, 0, 0, 0)),
            pl.BlockSpec((1, 16, D_MODEL), lambda b, q, h: (h, 0, 0)),
            pl.BlockSpec((1, 16, 1), lambda b, q, h: (h, 0, 0)),
            pl.BlockSpec((16, 1), lambda b, q, h: (0, 0)),
            pl.BlockSpec((16, 1), lambda b, q, h: (0, 0)),
            pl.BlockSpec((16, 1), lambda b, q, h: (0, 0)),
        ],
        out_specs=[
            pl.BlockSpec((1, 1, 16, QBLK_B), lambda b, q, h: (b, h, 0, q)),
            pl.BlockSpec((1, 1, 4, 16, QBLK_B), lambda b, q, h: (b, h, 0, 0, q)),
            pl.BlockSpec((1, 1, 4, 16, QBLK_B), lambda b, q, h: (b, h, 0, 0, q)),
        ],
        out_shape=[
            jax.ShapeDtypeStruct((B, N_HEADS, 16, Q), jnp.float32),
            jax.ShapeDtypeStruct((B, N_HEADS, 4, 16, Q), jnp.int32),
            jax.ShapeDtypeStruct((B, N_HEADS, 4, 16, Q), jnp.float32),
        ],
    )(hsT, refT, W_offT, b_offT, W_attnT, b_attnT, wl, hl, ls)


def _sc_gather(value4, idx4, w4):
    B = value4.shape[0]
    S = value4.shape[2] // 16
    Q = idx4.shape[-1]
    assert Q % QC == 0
    nfull = Q // QC
    mesh = plsc.VectorSubcoreMesh(core_axis_name="c", subcore_axis_name="s")

    @functools.partial(
        pl.kernel,
        out_type=jax.ShapeDtypeStruct((B, HK, 16, Q), jnp.float32),
        mesh=mesh,
        compiler_params=pltpu.CompilerParams(needs_layout_passes=False),
        scratch_types=[
            pltpu.VMEM((16 * S,), jnp.float32),
            pltpu.VMEM((2, 4, 16, QC), jnp.int32),
            pltpu.VMEM((4, 16, QC), jnp.float32),
            pltpu.VMEM((16, QC), jnp.float32),
            pltpu.SemaphoreType.DMA((2,)),
        ],
    )
    def body(value_hbm, idx_hbm, w_hbm, out_hbm, val_v, idx_v, w_v, out_v, si):
        b = lax.axis_index("c")
        s = lax.axis_index("s")
        h = s // 2
        pltpu.sync_copy(value_hbm.at[b, s], val_v)
        views = [val_v.at[pl.ds(d * S, S)] for d in range(16)]

        def start_in(k, par):
            pltpu.async_copy(idx_hbm.at[b, h, :, :, pl.ds(k * QC, QC)],
                             idx_v.at[par], si.at[par])

        def wait_in(k, par):
            pltpu.make_async_copy(idx_hbm.at[b, h, :, :, pl.ds(k * QC, QC)],
                                  idx_v.at[par], si.at[par]).wait()

        start_in(0, 0)

        def chunk_body(k):
            par = k % 2
            q0 = k * QC

            @pl.when(k + 1 < nfull)
            def _():
                start_in(k + 1, 1 - par)
            pltpu.sync_copy(w_hbm.at[b, h, :, :, pl.ds(q0, QC)], w_v)
            wait_in(k, par)

            def subblock(j):
                js = j * 16
                acc = [jnp.zeros((16,), jnp.float32) for _ in range(16)]
                for c in range(4):
                    for t in range(16):
                        rows = idx_v[par, c, t, pl.ds(js, 16)]
                        wv = w_v[c, t, pl.ds(js, 16)]
                        for d in range(16):
                            g = plsc.load_gather(views[d], [rows])
                            acc[d] = acc[d] + wv * g
                for d in range(16):
                    out_v[d, pl.ds(js, 16)] = acc[d]

            pl.loop(0, NSUB)(subblock)
            pltpu.sync_copy(out_v, out_hbm.at[b, s, :, pl.ds(q0, QC)])

        pl.loop(0, nfull)(chunk_body)

    return body(value4, idx4, w4)


def _out_proj_body(sc_ref, wo_ref, bo_ref, out_ref):
    x = sc_ref[0].reshape(D_MODEL, QBLK_D)
    y = lax.dot_general(x, wo_ref[...], (((0,), (0,)), ((), ())),
                        preferred_element_type=jnp.float32,
                        precision=jax.lax.Precision.HIGHEST)
    out_ref[0] = y + bo_ref[...]


def _out_proj(sc_out, W_out, b_out):
    B = sc_out.shape[0]
    Q = sc_out.shape[3]
    assert Q == QBLK_D
    grid = (B, Q // QBLK_D)
    return pl.pallas_call(
        _out_proj_body,
        grid=grid,
        in_specs=[
            pl.BlockSpec((1, HK, 16, QBLK_D), lambda b, q: (b, 0, 0, q)),
            pl.BlockSpec((D_MODEL, D_MODEL), lambda b, q: (0, 0)),
            pl.BlockSpec((D_MODEL,), lambda b, q: (0,)),
        ],
        out_specs=pl.BlockSpec((1, QBLK_D, D_MODEL), lambda b, q: (b, q, 0)),
        out_shape=jax.ShapeDtypeStruct((B, Q, D_MODEL), jnp.float32),
    )(sc_out, W_out, b_out)


def kernel(hidden_states, encoder_hidden_states, reference_points,
           spatial_shapes, level_start_index, W_value, b_value, W_off, b_off,
           W_attn, b_attn, W_out, b_out):
    B, Q, d = hidden_states.shape
    S = encoder_hidden_states.shape[1]
    H, L, P = N_HEADS, N_LEVELS, N_POINTS

    pad = QPAD - Q
    hsT = jnp.swapaxes(hidden_states, 1, 2)
    hsT = jnp.pad(hsT, ((0, 0), (0, 0), (0, pad)))
    refT = jnp.transpose(reference_points, (0, 3, 2, 1))
    refT = jnp.repeat(refT, N_POINTS, axis=2)
    refT = jnp.pad(refT, ((0, 0), (0, 0), (0, 0), (0, pad)))
    W_offT = jnp.transpose(W_off.reshape(d, H, 16, 2), (1, 3, 2, 0))
    b_offT = jnp.transpose(b_off.reshape(H, 16, 2), (0, 2, 1))[..., None]
    W_attnT = jnp.transpose(W_attn.reshape(d, H, 16), (1, 2, 0))
    b_attnT = b_attn.reshape(H, 16)[..., None]

    lp = jnp.arange(16)
    lvl = lp // N_POINTS
    sh = jnp.array(SPATIAL, jnp.float32)
    wl = sh[:, 1][lvl][:, None]
    hl = sh[:, 0][lvl][:, None]
    sizes = jnp.array([hh * ww for hh, ww in SPATIAL], jnp.float32)
    starts = jnp.concatenate([jnp.zeros((1,), jnp.float32), jnp.cumsum(sizes)[:-1]])
    ls = starts[lvl][:, None]

    value = _value_proj(encoder_hidden_states, W_value, b_value)
    value4 = jnp.transpose(value.reshape(B, S, HK, 16), (0, 2, 3, 1)).reshape(B, HK, 16 * S)

    attnT, idx4, w4 = _build_samples(hsT, refT, W_offT, b_offT,
                                     W_attnT, b_attnT, wl, hl, ls)

    sc_out = _sc_gather(value4, idx4, w4)

    out = _out_proj(sc_out, W_out, b_out)[:, :Q]

    attn = jnp.transpose(attnT[..., :Q], (0, 3, 1, 2)).reshape(B, Q, H, L, P)
    return out, attn
